# Initial kernel scaffold; baseline (speedup 1.0000x reference)
import jax
import jax.numpy as jnp
from jax.experimental import pallas as pl


def kernel(x, edge_index, W1, b1, W2, b2):
    raise NotImplementedError("write your pallas kernel here")


# speedup vs baseline: 12.9661x; 12.9661x over previous
import functools

import jax
import jax.numpy as jnp
from jax import lax
from jax.experimental import pallas as pl
from jax.experimental.pallas import tpu as pltpu, tpu_sc as plsc

NC = 2
NS = 16
LANES = 128
F32 = jnp.float32

_mesh = plsc.VectorSubcoreMesh(
    core_axis_name="c", subcore_axis_name="s", num_cores=NC, num_subcores=NS
)


def _deg_call(dst3, np_rows, chunks):
    rows_per_tile = np_rows // NS

    @functools.partial(
        pl.kernel,
        out_type=jax.ShapeDtypeStruct((NC, np_rows, 16), F32),
        mesh=_mesh,
        scratch_types=[
            pltpu.VMEM((chunks, LANES), jnp.int32),
            pltpu.VMEM((LANES, 16), F32),
            pltpu.VMEM((16, 16), F32),
            pltpu.VMEM_SHARED((np_rows, 16), F32),
        ],
    )
    def deg_kernel(dst_hbm, out_hbm, idx_v, ones_v, zeros_v, deg_sh):
        c = lax.axis_index("c")
        s = lax.axis_index("s")
        wid = c * NS + s
        for r in range(16):
            zeros_v[r] = jnp.zeros((16,), F32)
        for r in range(LANES):
            ones_v[r] = jnp.ones((16,), F32)
        r0 = s * rows_per_tile

        def zbody(i, carry):
            pltpu.sync_copy(zeros_v, deg_sh.at[pl.ds(r0 + i * 16, 16)])
            return carry

        lax.fori_loop(0, rows_per_tile // 16, zbody, None)
        plsc.subcore_barrier()
        pltpu.sync_copy(dst_hbm.at[wid], idx_v)

        def body(j, carry):
            pltpu.sync_copy(ones_v, deg_sh.at[idx_v.at[j]], add=True)
            return carry

        lax.fori_loop(0, chunks, body, None)
        plsc.subcore_barrier()
        pltpu.sync_copy(
            deg_sh.at[pl.ds(r0, rows_per_tile)],
            out_hbm.at[c, pl.ds(r0, rows_per_tile)],
        )

    return deg_kernel(dst3)


def _scatter_call(y, src3, dst3, np_rows, chunks):
    rows_per_tile = np_rows // NS

    @functools.partial(
        pl.kernel,
        out_type=jax.ShapeDtypeStruct((NC, np_rows, LANES), F32),
        mesh=_mesh,
        scratch_types=[
            pltpu.VMEM((chunks, LANES), jnp.int32),
            pltpu.VMEM((chunks, LANES), jnp.int32),
            pltpu.VMEM((LANES, LANES), F32),
            pltpu.VMEM((16, LANES), F32),
            pltpu.VMEM_SHARED((np_rows, LANES), F32),
            pltpu.SemaphoreType.DMA,
        ],
    )
    def scatter_kernel(y_hbm, src_hbm, dst_hbm, out_hbm,
                       src_v, dst_v, rows_v, zeros_v, acc_sh, sem):
        c = lax.axis_index("c")
        s = lax.axis_index("s")
        wid = c * NS + s
        for r in range(16):
            for k in range(LANES // 16):
                zeros_v[r, pl.ds(k * 16, 16)] = jnp.zeros((16,), F32)
        r0 = s * rows_per_tile

        def zbody(i, carry):
            pltpu.sync_copy(zeros_v, acc_sh.at[pl.ds(r0 + i * 16, 16)])
            return carry

        lax.fori_loop(0, rows_per_tile // 16, zbody, None)
        plsc.subcore_barrier()
        pltpu.sync_copy(src_hbm.at[wid], src_v)
        pltpu.sync_copy(dst_hbm.at[wid], dst_v)

        def body(j, carry):
            pltpu.async_copy(y_hbm.at[src_v.at[j]], rows_v, sem).wait()
            pltpu.sync_copy(rows_v, acc_sh.at[dst_v.at[j]], add=True)
            return carry

        lax.fori_loop(0, chunks, body, None)
        plsc.subcore_barrier()
        pltpu.sync_copy(
            acc_sh.at[pl.ds(r0, rows_per_tile)],
            out_hbm.at[c, pl.ds(r0, rows_per_tile)],
        )

    return scatter_kernel(y, src3, dst3)


def _dinv(d0_ref, d1_ref):
    deg = d0_ref[:, 0:1] + d1_ref[:, 0:1] + 1.0
    return lax.rsqrt(deg)


def _tc_layer1(xp, W1, d0, d1, np_rows, br):
    def body(x_ref, w_ref, d0_ref, d1_ref, y_ref):
        dinv = _dinv(d0_ref, d1_ref)
        y_ref[...] = dinv * jnp.dot(x_ref[...], w_ref[...],
                                    preferred_element_type=F32)

    grid = (np_rows // br,)
    return pl.pallas_call(
        body,
        grid=grid,
        in_specs=[
            pl.BlockSpec((br, LANES), lambda i: (i, 0)),
            pl.BlockSpec((LANES, LANES), lambda i: (0, 0)),
            pl.BlockSpec((br, 16), lambda i: (i, 0)),
            pl.BlockSpec((br, 16), lambda i: (i, 0)),
        ],
        out_specs=pl.BlockSpec((br, LANES), lambda i: (i, 0)),
        out_shape=jax.ShapeDtypeStruct((np_rows, LANES), F32),
    )(xp, W1, d0, d1)


def _tc_layer2(a0, a1, y1, d0, d1, W2, b1r, np_rows, br):
    def body(a0_ref, a1_ref, y_ref, d0_ref, d1_ref, w_ref, b_ref, o_ref):
        dinv = _dinv(d0_ref, d1_ref)
        pre = dinv * (a0_ref[...] + a1_ref[...] + y_ref[...]) + b_ref[...]
        h = jnp.where(pre > 0, pre, jnp.exp(pre) - 1.0)
        o_ref[...] = dinv * jnp.dot(h, w_ref[...], preferred_element_type=F32)

    grid = (np_rows // br,)
    return pl.pallas_call(
        body,
        grid=grid,
        in_specs=[
            pl.BlockSpec((br, LANES), lambda i: (i, 0)),
            pl.BlockSpec((br, LANES), lambda i: (i, 0)),
            pl.BlockSpec((br, LANES), lambda i: (i, 0)),
            pl.BlockSpec((br, 16), lambda i: (i, 0)),
            pl.BlockSpec((br, 16), lambda i: (i, 0)),
            pl.BlockSpec((LANES, LANES), lambda i: (0, 0)),
            pl.BlockSpec((1, LANES), lambda i: (0, 0)),
        ],
        out_specs=pl.BlockSpec((br, LANES), lambda i: (i, 0)),
        out_shape=jax.ShapeDtypeStruct((np_rows, LANES), F32),
    )(a0, a1, y1, d0, d1, W2, b1r)


def _tc_out(a0, a1, y2, d0, d1, b2r, np_rows, br):
    def body(a0_ref, a1_ref, y_ref, d0_ref, d1_ref, b_ref, o_ref):
        dinv = _dinv(d0_ref, d1_ref)
        o_ref[...] = dinv * (a0_ref[...] + a1_ref[...] + y_ref[...]) + b_ref[...]

    grid = (np_rows // br,)
    return pl.pallas_call(
        body,
        grid=grid,
        in_specs=[
            pl.BlockSpec((br, LANES), lambda i: (i, 0)),
            pl.BlockSpec((br, LANES), lambda i: (i, 0)),
            pl.BlockSpec((br, LANES), lambda i: (i, 0)),
            pl.BlockSpec((br, 16), lambda i: (i, 0)),
            pl.BlockSpec((br, 16), lambda i: (i, 0)),
            pl.BlockSpec((1, LANES), lambda i: (0, 0)),
        ],
        out_specs=pl.BlockSpec((br, LANES), lambda i: (i, 0)),
        out_shape=jax.ShapeDtypeStruct((np_rows, LANES), F32),
    )(a0, a1, y2, d0, d1, b2r)


def kernel(x, edge_index, W1, b1, W2, b2):
    n, d = x.shape
    e = edge_index.shape[1]
    np_rows = ((n + 1023) // 1024) * 1024
    nt = NC * NS
    chunks = (e + nt * LANES - 1) // (nt * LANES)
    ep = nt * chunks * LANES

    ei = edge_index.astype(jnp.int32)
    pad = jnp.full((ep - e,), n, jnp.int32)
    src3 = jnp.concatenate([ei[0], pad]).reshape(nt, chunks, LANES)
    dst3 = jnp.concatenate([ei[1], pad]).reshape(nt, chunks, LANES)
    xp = jnp.pad(x, ((0, np_rows - n), (0, 0)))
    b1r = b1.reshape(1, d)
    b2r = b2.reshape(1, d)

    br = 1024
    degp = _deg_call(dst3, np_rows, chunks)
    d0, d1 = degp[0], degp[1]
    y1 = _tc_layer1(xp, W1, d0, d1, np_rows, br)
    acc1 = _scatter_call(y1, src3, dst3, np_rows, chunks)
    y2 = _tc_layer2(acc1[0], acc1[1], y1, d0, d1, W2, b1r, np_rows, br)
    acc2 = _scatter_call(y2, src3, dst3, np_rows, chunks)
    out = _tc_out(acc2[0], acc2[1], y2, d0, d1, b2r, np_rows, br)
    return out[:n]
